# Initial kernel scaffold; baseline (speedup 1.0000x reference)
import jax
import jax.numpy as jnp
from jax.experimental import pallas as pl


def kernel(clf_preds, reg_preds, priors, gt_boxes, gt_labels):
    raise NotImplementedError("write your pallas kernel here")


# speedup vs baseline: 29.9316x; 29.9316x over previous
import functools

import jax
import jax.numpy as jnp
from jax.experimental import pallas as pl
from jax.experimental.pallas import tpu as pltpu

_NEG_RATIO = 3
_N_CLASSES = 21
_IOU_THRESH = 0.5
_B, _P, _M = 16, 32768, 32
_SL, _LN = 256, 128


def _loss_kernel(clf_ref, reg_ref, priors_ref, boxes_ref, labels_ref, out_ref):
    img = pl.program_id(0)

    px1 = priors_ref[0]
    py1 = priors_ref[1]
    px2 = priors_ref[2]
    py2 = priors_ref[3]
    parea = (px2 - px1) * (py2 - py1)

    iota_p = (
        jax.lax.broadcasted_iota(jnp.int32, (_SL, _LN), 0) * _LN
        + jax.lax.broadcasted_iota(jnp.int32, (_SL, _LN), 1)
    )

    def match_body(m, carry):
        best_iou, mlab, mb0, mb1, mb2, mb3 = carry
        bx1 = boxes_ref[0, m, 0]
        by1 = boxes_ref[0, m, 1]
        bx2 = boxes_ref[0, m, 2]
        by2 = boxes_ref[0, m, 3]
        barea = (bx2 - bx1) * (by2 - by1)
        iw = jnp.maximum(jnp.minimum(px2, bx2) - jnp.maximum(px1, bx1), 0.0)
        ih = jnp.maximum(jnp.minimum(py2, by2) - jnp.maximum(py1, by1), 0.0)
        inter = iw * ih
        iou = inter / (parea + barea - inter + 1e-9)
        rmax = jnp.max(iou)
        pmin = jnp.min(jnp.where(iou == rmax, iota_p, _P))
        iou = jnp.where(iota_p == pmin, 1.0, iou)
        upd = iou > best_iou
        best_iou = jnp.where(upd, iou, best_iou)
        labf = labels_ref[0, 0, m].astype(jnp.float32)
        mlab = jnp.where(upd, labf, mlab)
        mb0 = jnp.where(upd, bx1, mb0)
        mb1 = jnp.where(upd, by1, mb1)
        mb2 = jnp.where(upd, bx2, mb2)
        mb3 = jnp.where(upd, by2, mb3)
        return best_iou, mlab, mb0, mb1, mb2, mb3

    zeros = jnp.zeros((_SL, _LN), jnp.float32)
    best_iou, mlab, mb0, mb1, mb2, mb3 = jax.lax.fori_loop(
        0, _M, match_body, (jnp.full((_SL, _LN), -1.0, jnp.float32),
                            zeros, zeros, zeros, zeros, zeros))

    mlab = jnp.where(best_iou > _IOU_THRESH, mlab, 0.0)
    pos = mlab > 0.0
    n_pos = jnp.sum(pos.astype(jnp.int32))
    k_neg = _NEG_RATIO * n_pos

    clf = clf_ref[0]
    maxc = jnp.max(clf, axis=0)
    es = jnp.sum(jnp.exp(clf - maxc[None, :, :]), axis=0)
    lse = maxc + jnp.log(es)
    iota_c = jax.lax.broadcasted_iota(jnp.int32, (_N_CLASSES, _SL, _LN), 0)
    mlab_i = mlab.astype(jnp.int32)
    cmatch = jnp.sum(jnp.where(iota_c == mlab_i[None, :, :], clf, 0.0), axis=0)
    nll = lse - cmatch
    pos_sum = jnp.sum(jnp.where(pos, nll, 0.0))

    bg = jnp.where(pos, 0.0, lse - clf[0])

    reg = reg_ref[0]
    se = ((reg[0] - mb0) ** 2 + (reg[1] - mb1) ** 2
          + (reg[2] - mb2) ** 2 + (reg[3] - mb3) ** 2)
    reg_sum = jnp.sum(jnp.where(pos, se, 0.0))

    v = jax.lax.bitcast_convert_type(bg, jnp.int32)

    def select_body(i, prefix):
        cand = prefix | jax.lax.shift_left(jnp.int32(1), 30 - i)
        cnt = jnp.sum((v >= cand).astype(jnp.int32))
        return jnp.where(cnt >= k_neg, cand, prefix)

    prefix = jax.lax.fori_loop(0, 31, select_body, jnp.int32(0))
    gt_mask = v > prefix
    cnt_gt = jnp.sum(gt_mask.astype(jnp.int32))
    sum_gt = jnp.sum(jnp.where(gt_mask, bg, 0.0))
    t_val = jax.lax.bitcast_convert_type(prefix, jnp.float32)
    tie_cnt = (k_neg - cnt_gt).astype(jnp.float32)
    neg_sum = sum_gt + jnp.where(k_neg > cnt_gt, tie_cnt * t_val, 0.0)

    n_posf = n_pos.astype(jnp.float32)
    clf_l = (pos_sum / jnp.maximum(n_posf, 1.0)
             + neg_sum / jnp.maximum(_NEG_RATIO * n_posf, 1.0))
    reg_l = reg_sum / jnp.maximum(4.0 * n_posf, 1.0)

    @pl.when(img == 0)
    def _():
        out_ref[0, 0] = 0.0

    out_ref[0, 0] += (clf_l + reg_l) * (1.0 / _B)


@jax.jit
def kernel(clf_preds, reg_preds, priors, gt_boxes, gt_labels):
    clf_t = clf_preds.transpose(0, 2, 1).reshape(_B, _N_CLASSES, _SL, _LN)
    reg_t = reg_preds.transpose(0, 2, 1).reshape(_B, 4, _SL, _LN)
    priors_t = priors.T.reshape(4, _SL, _LN)

    out = pl.pallas_call(
        _loss_kernel,
        grid=(_B,),
        in_specs=[
            pl.BlockSpec((1, _N_CLASSES, _SL, _LN), lambda i: (i, 0, 0, 0)),
            pl.BlockSpec((1, 4, _SL, _LN), lambda i: (i, 0, 0, 0)),
            pl.BlockSpec((4, _SL, _LN), lambda i: (0, 0, 0)),
            pl.BlockSpec((1, _M, 4), lambda i: (i, 0, 0),
                         memory_space=pltpu.SMEM),
            pl.BlockSpec((1, 1, _M), lambda i: (i, 0, 0),
                         memory_space=pltpu.SMEM),
        ],
        out_specs=pl.BlockSpec((1, 1), lambda i: (0, 0),
                               memory_space=pltpu.SMEM),
        out_shape=jax.ShapeDtypeStruct((1, 1), jnp.float32),
    )(clf_t, reg_t, priors_t, gt_boxes, gt_labels.reshape(_B, 1, _M))
    return out[0, 0]


# speedup vs baseline: 42.8301x; 1.4309x over previous
import functools

import jax
import jax.numpy as jnp
from jax.experimental import pallas as pl
from jax.experimental.pallas import tpu as pltpu

_NEG_RATIO = 3
_N_CLASSES = 21
_IOU_THRESH = 0.5
_B, _P, _M = 16, 32768, 32
_SL, _LN = 256, 128


def _loss_kernel(clf_ref, reg_ref, priors_ref, boxes_ref, labels_ref, out_ref):
    img = pl.program_id(0)

    px1 = priors_ref[0]
    py1 = priors_ref[1]
    px2 = priors_ref[2]
    py2 = priors_ref[3]
    parea = (px2 - px1) * (py2 - py1)

    iota_p = (
        jax.lax.broadcasted_iota(jnp.int32, (_SL, _LN), 0) * _LN
        + jax.lax.broadcasted_iota(jnp.int32, (_SL, _LN), 1)
    )

    def match_body(m, carry):
        best_iou, mlab, mb0, mb1, mb2, mb3 = carry
        bx1 = boxes_ref[0, m, 0]
        by1 = boxes_ref[0, m, 1]
        bx2 = boxes_ref[0, m, 2]
        by2 = boxes_ref[0, m, 3]
        barea = (bx2 - bx1) * (by2 - by1)
        iw = jnp.maximum(jnp.minimum(px2, bx2) - jnp.maximum(px1, bx1), 0.0)
        ih = jnp.maximum(jnp.minimum(py2, by2) - jnp.maximum(py1, by1), 0.0)
        inter = iw * ih
        iou = inter / (parea + barea - inter + 1e-9)
        rmax = jnp.max(iou)
        pmin = jnp.min(jnp.where(iou == rmax, iota_p, _P))
        iou = jnp.where(iota_p == pmin, 1.0, iou)
        upd = iou > best_iou
        best_iou = jnp.where(upd, iou, best_iou)
        labf = labels_ref[0, 0, m].astype(jnp.float32)
        mlab = jnp.where(upd, labf, mlab)
        mb0 = jnp.where(upd, bx1, mb0)
        mb1 = jnp.where(upd, by1, mb1)
        mb2 = jnp.where(upd, bx2, mb2)
        mb3 = jnp.where(upd, by2, mb3)
        return best_iou, mlab, mb0, mb1, mb2, mb3

    zeros = jnp.zeros((_SL, _LN), jnp.float32)
    best_iou, mlab, mb0, mb1, mb2, mb3 = jax.lax.fori_loop(
        0, _M, match_body, (jnp.full((_SL, _LN), -1.0, jnp.float32),
                            zeros, zeros, zeros, zeros, zeros),
        unroll=4)

    mlab = jnp.where(best_iou > _IOU_THRESH, mlab, 0.0)
    pos = mlab > 0.0
    n_pos = jnp.sum(pos.astype(jnp.int32))
    k_neg = _NEG_RATIO * n_pos

    mlab_i = mlab.astype(jnp.int32)
    c0 = clf_ref[0, 0]
    maxc = c0
    for c in range(1, _N_CLASSES):
        maxc = jnp.maximum(maxc, clf_ref[0, c])
    es = jnp.exp(c0 - maxc)
    cmatch = jnp.where(mlab_i == 0, c0, 0.0)
    for c in range(1, _N_CLASSES):
        x = clf_ref[0, c]
        es += jnp.exp(x - maxc)
        cmatch = jnp.where(mlab_i == c, x, cmatch)
    lse = maxc + jnp.log(es)
    nll = lse - cmatch
    pos_sum = jnp.sum(jnp.where(pos, nll, 0.0))

    bg = jnp.where(pos, 0.0, lse - c0)

    se = ((reg_ref[0, 0] - mb0) ** 2 + (reg_ref[0, 1] - mb1) ** 2
          + (reg_ref[0, 2] - mb2) ** 2 + (reg_ref[0, 3] - mb3) ** 2)
    reg_sum = jnp.sum(jnp.where(pos, se, 0.0))

    v = jax.lax.bitcast_convert_type(bg, jnp.int32)

    prefix = jnp.int32(0)
    cand = jnp.int32(1 << 30)
    cnt = jnp.sum((v >= cand).astype(jnp.int32))
    prefix = jnp.where(cnt >= k_neg, cand, prefix)
    for h in range(28, -2, -2):
        c1 = prefix | jnp.int32(1 << h)
        c2 = prefix | jnp.int32(2 << h)
        c3 = prefix | jnp.int32(3 << h)
        n1 = jnp.sum((v >= c1).astype(jnp.int32))
        n2 = jnp.sum((v >= c2).astype(jnp.int32))
        n3 = jnp.sum((v >= c3).astype(jnp.int32))
        prefix = jnp.where(
            n3 >= k_neg, c3,
            jnp.where(n2 >= k_neg, c2, jnp.where(n1 >= k_neg, c1, prefix)))
    gt_mask = v > prefix
    cnt_gt = jnp.sum(gt_mask.astype(jnp.int32))
    sum_gt = jnp.sum(jnp.where(gt_mask, bg, 0.0))
    t_val = jax.lax.bitcast_convert_type(prefix, jnp.float32)
    tie_cnt = (k_neg - cnt_gt).astype(jnp.float32)
    neg_sum = sum_gt + jnp.where(k_neg > cnt_gt, tie_cnt * t_val, 0.0)

    n_posf = n_pos.astype(jnp.float32)
    clf_l = (pos_sum / jnp.maximum(n_posf, 1.0)
             + neg_sum / jnp.maximum(_NEG_RATIO * n_posf, 1.0))
    reg_l = reg_sum / jnp.maximum(4.0 * n_posf, 1.0)

    @pl.when(img == 0)
    def _():
        out_ref[0, 0] = 0.0

    out_ref[0, 0] += (clf_l + reg_l) * (1.0 / _B)


@jax.jit
def kernel(clf_preds, reg_preds, priors, gt_boxes, gt_labels):
    clf_t = clf_preds.transpose(0, 2, 1).reshape(_B, _N_CLASSES, _SL, _LN)
    reg_t = reg_preds.transpose(0, 2, 1).reshape(_B, 4, _SL, _LN)
    priors_t = priors.T.reshape(4, _SL, _LN)

    out = pl.pallas_call(
        _loss_kernel,
        grid=(_B,),
        in_specs=[
            pl.BlockSpec((1, _N_CLASSES, _SL, _LN), lambda i: (i, 0, 0, 0)),
            pl.BlockSpec((1, 4, _SL, _LN), lambda i: (i, 0, 0, 0)),
            pl.BlockSpec((4, _SL, _LN), lambda i: (0, 0, 0)),
            pl.BlockSpec((1, _M, 4), lambda i: (i, 0, 0),
                         memory_space=pltpu.SMEM),
            pl.BlockSpec((1, 1, _M), lambda i: (i, 0, 0),
                         memory_space=pltpu.SMEM),
        ],
        out_specs=pl.BlockSpec((1, 1), lambda i: (0, 0),
                               memory_space=pltpu.SMEM),
        out_shape=jax.ShapeDtypeStruct((1, 1), jnp.float32),
    )(clf_t, reg_t, priors_t, gt_boxes, gt_labels.reshape(_B, 1, _M))
    return out[0, 0]
